# baseline (speedup 1.0000x reference)
import jax
import jax.numpy as jnp
from jax.experimental import pallas as pl

_SIZE = 65536
_SHIFT = 8192
_ROWS = 64
_NB = _SIZE // _SHIFT


def _body(x_ref, o_ref):
    j = pl.program_id(0)

    @pl.when(j < _NB - 1)
    def _zero():
        o_ref[...] = jnp.zeros_like(o_ref)

    @pl.when(j == _NB - 1)
    def _copy():
        o_ref[...] = x_ref[...]


def kernel(x):
    xf = x.reshape(_ROWS, _SHIFT)
    out = pl.pallas_call(
        _body,
        grid=(_NB,),
        in_specs=[pl.BlockSpec((_ROWS, _SHIFT), lambda j: (0, 0))],
        out_specs=pl.BlockSpec((_ROWS, _SHIFT), lambda j: (0, j)),
        out_shape=jax.ShapeDtypeStruct((_ROWS, _SIZE), jnp.float32),
    )(xf)
    return out.reshape(x.shape[:-1] + (_SIZE,))
